# Initial kernel scaffold; baseline (speedup 1.0000x reference)
import jax
import jax.numpy as jnp
from jax.experimental import pallas as pl


def kernel(hidden_states, weight):
    raise NotImplementedError("write your pallas kernel here")


# speedup vs baseline: 1.4120x; 1.4120x over previous
import jax
import jax.numpy as jnp
from jax.experimental import pallas as pl

_NUM_EXPERTS = 64
_TOP_K = 8
_BT = 1024


def _router_kernel(x_ref, wt_ref, logits_ref, vals_ref, idx_ref):
    x = x_ref[...]
    wt = wt_ref[...]
    logits = jnp.dot(x, wt, preferred_element_type=jnp.float32)
    logits_ref[...] = logits

    work = logits
    lane = jax.lax.broadcasted_iota(jnp.int32, work.shape, 1)
    vals = []
    idxs = []
    for _ in range(_TOP_K):
        m = jnp.max(work, axis=-1, keepdims=True)
        i = jnp.argmax(work, axis=-1).astype(jnp.int32)
        vals.append(m)
        idxs.append(i[:, None])
        work = jnp.where(lane == i[:, None], -jnp.inf, work)
    top_vals = jnp.concatenate(vals, axis=-1)
    top_idx = jnp.concatenate(idxs, axis=-1)
    e = jnp.exp(top_vals - top_vals[:, 0:1])
    vals_ref[...] = e / jnp.sum(e, axis=-1, keepdims=True)
    idx_ref[...] = top_idx


@jax.jit
def kernel(hidden_states, weight):
    tokens, hidden = hidden_states.shape
    wt = weight.T
    grid = (tokens // _BT,)
    out = pl.pallas_call(
        _router_kernel,
        grid=grid,
        in_specs=[
            pl.BlockSpec((_BT, hidden), lambda i: (i, 0)),
            pl.BlockSpec((hidden, _NUM_EXPERTS), lambda i: (0, 0)),
        ],
        out_specs=[
            pl.BlockSpec((_BT, _NUM_EXPERTS), lambda i: (i, 0)),
            pl.BlockSpec((_BT, _TOP_K), lambda i: (i, 0)),
            pl.BlockSpec((_BT, _TOP_K), lambda i: (i, 0)),
        ],
        out_shape=[
            jax.ShapeDtypeStruct((tokens, _NUM_EXPERTS), jnp.float32),
            jax.ShapeDtypeStruct((tokens, _TOP_K), jnp.float32),
            jax.ShapeDtypeStruct((tokens, _TOP_K), jnp.int32),
        ],
    )(hidden_states, wt)
    return tuple(out)


# speedup vs baseline: 1.6889x; 1.1961x over previous
import jax
import jax.numpy as jnp
from jax.experimental import pallas as pl

_NUM_EXPERTS = 64
_TOP_K = 8
_BT = 1024
_SUB = 512


def _router_kernel(x_ref, wt_ref, logits_ref, vals_ref, idx_ref):
    x = x_ref[...]
    wt = wt_ref[...]
    logits = jnp.dot(x, wt, preferred_element_type=jnp.float32)
    logits_ref[...] = logits

    for c in range(_BT // _SUB):
        cols = pl.ds(c * _SUB, _SUB)
        work = logits_ref[cols, :].T
        row = jax.lax.broadcasted_iota(jnp.int32, work.shape, 0)
        vals = []
        idxs = []
        for _ in range(_TOP_K):
            m = jnp.max(work, axis=0, keepdims=True)
            i = jnp.argmax(work, axis=0).astype(jnp.int32)[None, :]
            vals.append(m)
            idxs.append(i)
            work = jnp.where(row == i, -jnp.inf, work)
        top_vals = jnp.concatenate(vals, axis=0)
        top_idx = jnp.concatenate(idxs, axis=0)
        e = jnp.exp(top_vals - top_vals[0:1, :])
        vals_ref[cols, :] = (e / jnp.sum(e, axis=0, keepdims=True)).T
        idx_ref[cols, :] = top_idx.T


@jax.jit
def kernel(hidden_states, weight):
    tokens, hidden = hidden_states.shape
    wt = weight.T
    grid = (tokens // _BT,)
    out = pl.pallas_call(
        _router_kernel,
        grid=grid,
        in_specs=[
            pl.BlockSpec((_BT, hidden), lambda i: (i, 0)),
            pl.BlockSpec((hidden, _NUM_EXPERTS), lambda i: (0, 0)),
        ],
        out_specs=[
            pl.BlockSpec((_BT, _NUM_EXPERTS), lambda i: (i, 0)),
            pl.BlockSpec((_BT, _TOP_K), lambda i: (i, 0)),
            pl.BlockSpec((_BT, _TOP_K), lambda i: (i, 0)),
        ],
        out_shape=[
            jax.ShapeDtypeStruct((tokens, _NUM_EXPERTS), jnp.float32),
            jax.ShapeDtypeStruct((tokens, _TOP_K), jnp.float32),
            jax.ShapeDtypeStruct((tokens, _TOP_K), jnp.int32),
        ],
    )(hidden_states, wt)
    return tuple(out)


# speedup vs baseline: 1.7315x; 1.0253x over previous
import jax
import jax.numpy as jnp
from jax.experimental import pallas as pl

_NUM_EXPERTS = 64
_TOP_K = 8
_BT = 2048
_SUB = 512


def _router_kernel(x_ref, wt_ref, logits_ref, vals_ref, idx_ref):
    x = x_ref[...]
    wt = wt_ref[...]
    logits = jnp.dot(x, wt, preferred_element_type=jnp.float32)
    logits_ref[...] = logits

    for c in range(_BT // _SUB):
        cols = pl.ds(c * _SUB, _SUB)
        work = logits_ref[cols, :].T
        row = jax.lax.broadcasted_iota(jnp.int32, work.shape, 0)
        vals = []
        idxs = []
        for _ in range(_TOP_K):
            m = jnp.max(work, axis=0, keepdims=True)
            i = jnp.argmax(work, axis=0).astype(jnp.int32)[None, :]
            vals.append(m)
            idxs.append(i)
            work = jnp.where(row == i, -jnp.inf, work)
        top_vals = jnp.concatenate(vals, axis=0)
        top_idx = jnp.concatenate(idxs, axis=0)
        e = jnp.exp(top_vals - top_vals[0:1, :])
        vals_ref[cols, :] = (e / jnp.sum(e, axis=0, keepdims=True)).T
        idx_ref[cols, :] = top_idx.T


@jax.jit
def kernel(hidden_states, weight):
    tokens, hidden = hidden_states.shape
    wt = weight.T
    grid = (tokens // _BT,)
    out = pl.pallas_call(
        _router_kernel,
        grid=grid,
        in_specs=[
            pl.BlockSpec((_BT, hidden), lambda i: (i, 0)),
            pl.BlockSpec((hidden, _NUM_EXPERTS), lambda i: (0, 0)),
        ],
        out_specs=[
            pl.BlockSpec((_BT, _NUM_EXPERTS), lambda i: (i, 0)),
            pl.BlockSpec((_BT, _TOP_K), lambda i: (i, 0)),
            pl.BlockSpec((_BT, _TOP_K), lambda i: (i, 0)),
        ],
        out_shape=[
            jax.ShapeDtypeStruct((tokens, _NUM_EXPERTS), jnp.float32),
            jax.ShapeDtypeStruct((tokens, _TOP_K), jnp.float32),
            jax.ShapeDtypeStruct((tokens, _TOP_K), jnp.int32),
        ],
    )(hidden_states, wt)
    return tuple(out)
